# Initial kernel scaffold; baseline (speedup 1.0000x reference)
import jax
import jax.numpy as jnp
from jax.experimental import pallas as pl


def kernel(pred_scores, true_scores, batch_ids):
    raise NotImplementedError("write your pallas kernel here")


# speedup vs baseline: 7.4461x; 7.4461x over previous
import jax
import jax.numpy as jnp
from jax.experimental import pallas as pl
from jax.experimental.pallas import tpu as pltpu

N = 8192
NB = 16
TR = 256
CC = 256
NI = N // TR
NC = N // CC


def _body(p_r_ref, t_r_ref, b_r_ref, p_c_ref, t_c_ref, b_c_ref,
          out_ref, acc_ref, cnt_ref):
    i = pl.program_id(0)

    @pl.when(i == 0)
    def _init():
        acc_ref[...] = jnp.zeros_like(acc_ref)
        cnt_ref[...] = jnp.zeros_like(cnt_ref)

    b_r = b_r_ref[...]
    p_r = p_r_ref[...]
    t_r = t_r_ref[...]
    bmax_r = b_r[TR - 1, 0]
    ce = jnp.sum((b_c_ref[...] <= bmax_r).astype(jnp.int32))
    nchunks = (ce + CC - 1) // CC

    r0 = i * TR
    iota_r = jax.lax.broadcasted_iota(jnp.int32, (TR, 1), 0) + r0

    def chunk(j, acc_rows):
        c0 = j * CC
        p_c = p_c_ref[pl.ds(j, 1), :]
        t_c = t_c_ref[pl.ds(j, 1), :]
        b_c = b_c_ref[pl.ds(j, 1), :]
        iota_c = jax.lax.broadcasted_iota(jnp.int32, (1, CC), 1) + c0
        mask = (b_r == b_c) & (iota_r < iota_c)
        d = p_r - p_c
        sp_pos = jnp.maximum(d, 0.0) + jnp.log(1.0 + jnp.exp(-jnp.abs(d)))
        sp_neg = sp_pos - d
        a = jnp.minimum(sp_neg, 100.0)
        b = jnp.minimum(sp_pos, 100.0)
        loss = jnp.where(t_r > t_c, a,
                         jnp.where(t_r < t_c, b, 0.5 * (a + b)))
        return acc_rows + jnp.sum(jnp.where(mask, loss, 0.0), axis=1,
                                  keepdims=True)

    acc_rows = jax.lax.fori_loop(i, nchunks, chunk,
                                 jnp.zeros((TR, 1), jnp.float32))

    bins = jax.lax.broadcasted_iota(jnp.int32, (1, NB), 1)
    onehot = (b_r == bins)
    acc_ref[...] += jnp.sum(jnp.where(onehot, acc_rows, 0.0), axis=0,
                            keepdims=True)
    cnt_ref[...] += jnp.sum(onehot.astype(jnp.int32), axis=0, keepdims=True)

    @pl.when(i == NI - 1)
    def _final():
        nb = cnt_ref[...]
        pair_sums = acc_ref[...]
        num_pairs = (nb * (nb - 1)) >> 1
        safe = jnp.where(num_pairs > 0, num_pairs, 1).astype(jnp.float32)
        loss_b = jnp.where(nb >= 2, pair_sums / safe, 0.0)
        total = jnp.sum(loss_b, axis=1, keepdims=True)
        count = jnp.sum((nb >= 2).astype(jnp.int32), axis=1,
                        keepdims=True)
        out_ref[...] = jnp.where(
            count > 0, total / jnp.maximum(count, 1).astype(jnp.float32),
            jnp.float32(0.0))


def kernel(pred_scores, true_scores, batch_ids):
    b = batch_ids.astype(jnp.int32)
    out = pl.pallas_call(
        _body,
        grid=(NI,),
        in_specs=[
            pl.BlockSpec((TR, 1), lambda i: (i, 0)),
            pl.BlockSpec((TR, 1), lambda i: (i, 0)),
            pl.BlockSpec((TR, 1), lambda i: (i, 0)),
            pl.BlockSpec((NC, CC), lambda i: (0, 0)),
            pl.BlockSpec((NC, CC), lambda i: (0, 0)),
            pl.BlockSpec((NC, CC), lambda i: (0, 0)),
        ],
        out_specs=pl.BlockSpec((1, 1), lambda i: (0, 0)),
        out_shape=jax.ShapeDtypeStruct((1, 1), jnp.float32),
        scratch_shapes=[
            pltpu.VMEM((1, NB), jnp.float32),
            pltpu.VMEM((1, NB), jnp.int32),
        ],
    )(
        pred_scores.reshape(N, 1), true_scores.reshape(N, 1),
        b.reshape(N, 1),
        pred_scores.reshape(NC, CC), true_scores.reshape(NC, CC),
        b.reshape(NC, CC),
    )
    return out[0, 0]
